# baseline (speedup 1.0000x reference)
import functools

import jax
import jax.numpy as jnp
from jax import lax
from jax.experimental import pallas as pl
from jax.experimental.pallas import tpu as pltpu
from jax.experimental.pallas import tpu_sc as plsc

EMBED_D = 1728

NC = 2
NS = 16
NW = NC * NS

CH = 32


@functools.partial(jax.jit, static_argnums=(2, 3))
def _sc_embedding_lookup(ids_flat, table, b_per_w, nch):
    mesh = plsc.VectorSubcoreMesh(
        core_axis_name="c", subcore_axis_name="s", num_cores=NC, num_subcores=NS
    )
    out_rows = b_per_w * NW

    @functools.partial(
        pl.kernel,
        out_type=jax.ShapeDtypeStruct((out_rows, EMBED_D), jnp.float32),
        mesh=mesh,
        scratch_types=[
            pltpu.VMEM((nch, CH), jnp.int32),
            pltpu.VMEM((CH, EMBED_D), jnp.float32),
            pltpu.VMEM((CH, EMBED_D), jnp.float32),
            pltpu.SemaphoreType.DMA,
            pltpu.SemaphoreType.DMA,
            pltpu.SemaphoreType.DMA,
        ],
        compiler_params=pltpu.CompilerParams(use_tc_tiling_on_sc=False),
    )
    def k(idx_hbm, table_hbm, out_hbm, idx_v, rows0, rows1, gsem, ssem0, ssem1):
        wid = lax.axis_index("s") * NC + lax.axis_index("c")
        gbase = wid * b_per_w
        pltpu.sync_copy(idx_hbm.at[wid], idx_v)

        def body(i, _):
            c0 = i * 2
            c1 = c0 + 1
            pltpu.async_copy(table_hbm.at[idx_v.at[c0]], rows0, gsem).wait()
            s0 = pltpu.async_copy(
                rows0, out_hbm.at[pl.ds(gbase + c0 * CH, CH)], ssem0
            )
            pltpu.async_copy(table_hbm.at[idx_v.at[c1]], rows1, gsem).wait()
            s1 = pltpu.async_copy(
                rows1, out_hbm.at[pl.ds(gbase + c1 * CH, CH)], ssem1
            )
            s0.wait()
            s1.wait()
            return _

        lax.fori_loop(0, nch // 2, body, None)

    return k(ids_flat, table)


def kernel(input_ids, font_weights):
    bsz, seq = input_ids.shape
    b = bsz * seq
    b_per_w = b // NW
    nch = b_per_w // CH
    ids = input_ids.astype(jnp.int32).reshape(NW, nch, CH)
    out = _sc_embedding_lookup(ids, font_weights, b_per_w, nch)
    return out.reshape(bsz, seq, EMBED_D)


# speedup vs baseline: 1.0032x; 1.0032x over previous
import functools

import jax
import jax.numpy as jnp
from jax import lax
from jax.experimental import pallas as pl
from jax.experimental.pallas import tpu as pltpu
from jax.experimental.pallas import tpu_sc as plsc

EMBED_D = 1728

NC = 2
NS = 16
NW = NC * NS

CH = 32


@functools.partial(jax.jit, static_argnums=(2, 3))
def _sc_embedding_lookup(ids, table, b_per_w, nch):
    mesh = plsc.VectorSubcoreMesh(
        core_axis_name="c", subcore_axis_name="s", num_cores=NC, num_subcores=NS
    )
    out_rows = b_per_w * NW

    @functools.partial(
        pl.kernel,
        out_type=jax.ShapeDtypeStruct((out_rows, EMBED_D), jnp.float32),
        mesh=mesh,
        scratch_types=[
            pltpu.VMEM((nch, CH), jnp.int32),
            [pltpu.VMEM((CH, EMBED_D), jnp.float32) for _ in range(2)],
            [pltpu.SemaphoreType.DMA for _ in range(2)],
            [pltpu.SemaphoreType.DMA for _ in range(2)],
        ],
        compiler_params=pltpu.CompilerParams(use_tc_tiling_on_sc=False),
    )
    def k(idx_hbm, table_hbm, out_hbm, idx_v, rows, gsem, ssem):
        wid = lax.axis_index("s") * NC + lax.axis_index("c")
        gbase = wid * b_per_w
        pltpu.sync_copy(idx_hbm.at[wid], idx_v)

        def start_gather(c, h):
            pltpu.async_copy(table_hbm.at[idx_v.at[c]], rows[h], gsem[h])

        def wait_gather(h):
            pltpu.make_async_copy(
                table_hbm.at[pl.ds(0, CH)], rows[h], gsem[h]
            ).wait()

        def start_scatter(c, h):
            pltpu.async_copy(
                rows[h], out_hbm.at[pl.ds(gbase + c * CH, CH)], ssem[h]
            )

        def wait_scatter(h):
            pltpu.make_async_copy(
                rows[h], out_hbm.at[pl.ds(gbase, CH)], ssem[h]
            ).wait()

        def group(c, h, first=False, last=False):
            wait_gather(h)
            start_scatter(c, h)
            if not first:
                wait_scatter(1 - h)
            if not last:
                start_gather(c + 1, 1 - h)

        start_gather(0, 0)
        group(0, 0, first=True)

        def body(i, _):
            group(i * 2 + 1, 1)
            group(i * 2 + 2, 0)
            return _

        lax.fori_loop(0, (nch - 2) // 2, body, None)

        group(nch - 1, 1, last=True)
        wait_scatter(1)

    return k(ids, table)


def kernel(input_ids, font_weights):
    bsz, seq = input_ids.shape
    b = bsz * seq
    b_per_w = b // NW
    nch = b_per_w // CH
    ids = input_ids.astype(jnp.int32).reshape(NW, nch, CH)
    out = _sc_embedding_lookup(ids, font_weights, b_per_w, nch)
    return out.reshape(bsz, seq, EMBED_D)


# speedup vs baseline: 1.2325x; 1.2285x over previous
import functools

import jax
import jax.numpy as jnp
from jax import lax
from jax.experimental import pallas as pl
from jax.experimental.pallas import tpu as pltpu
from jax.experimental.pallas import tpu_sc as plsc

V = 23236
D = 1728
BSZ = 64
SEQ = 512
NC = 2
NS = 16
NW = NC * NS
F_PER = D // NW


@jax.jit
def _sc_lookup(ids, table_t):
    mesh = plsc.VectorSubcoreMesh(
        core_axis_name="c", subcore_axis_name="s", num_cores=NC, num_subcores=NS
    )

    @functools.partial(
        pl.kernel,
        out_type=jax.ShapeDtypeStruct((BSZ, D, SEQ), jnp.float32),
        mesh=mesh,
        scratch_types=[
            pltpu.VMEM((BSZ * SEQ,), jnp.int32),
            pltpu.VMEM((V,), jnp.float32),
            pltpu.VMEM((BSZ, SEQ), jnp.float32),
        ],
        compiler_params=pltpu.CompilerParams(use_tc_tiling_on_sc=True, needs_layout_passes=False),
    )
    def k(ids_hbm, table_hbm, out_hbm, ids_v, row_v, obuf):
        wid = lax.axis_index("s") * NC + lax.axis_index("c")
        f0 = wid * F_PER
        pltpu.sync_copy(ids_hbm, ids_v)

        def feature(fi, _):
            f = f0 + fi
            pltpu.sync_copy(table_hbm.at[f], row_v)

            def per_b(b, _):
                def per_g(g, _):
                    base = b * SEQ + g * 16
                    idx = ids_v[pl.ds(base, 16)]
                    obuf[b, pl.ds(g * 16, 16)] = plsc.load_gather(row_v, [idx])
                    return _

                return lax.fori_loop(0, SEQ // 16, per_g, _)

            lax.fori_loop(0, BSZ, per_b, None)
            pltpu.sync_copy(obuf, out_hbm.at[:, f])
            return _

        lax.fori_loop(0, F_PER, feature, None)

    return k(ids, table_t)


def kernel(input_ids, font_weights):
    ids = input_ids.reshape(-1).astype(jnp.int32)
    table_t = font_weights.T
    out_t = _sc_lookup(ids, table_t)
    return jnp.transpose(out_t, (0, 2, 1))


# speedup vs baseline: 1.4395x; 1.1680x over previous
import functools

import jax
import jax.numpy as jnp
from jax import lax
from jax.experimental import pallas as pl
from jax.experimental.pallas import tpu as pltpu
from jax.experimental.pallas import tpu_sc as plsc

V = 23236
D = 1728
BSZ = 64
SEQ = 512
NC = 2
NS = 16
NW = NC * NS
F_PER = D // NW
HB = BSZ // 2


@jax.jit
def _sc_lookup(ids, table_t):
    mesh = plsc.VectorSubcoreMesh(
        core_axis_name="c", subcore_axis_name="s", num_cores=NC, num_subcores=NS
    )

    @functools.partial(
        pl.kernel,
        out_type=jax.ShapeDtypeStruct((BSZ, D, SEQ), jnp.float32),
        mesh=mesh,
        scratch_types=[
            pltpu.VMEM((BSZ * SEQ,), jnp.int32),
            [pltpu.VMEM((V,), jnp.float32) for _ in range(2)],
            [pltpu.VMEM((HB, SEQ), jnp.float32) for _ in range(2)],
            [pltpu.SemaphoreType.DMA for _ in range(2)],
            [pltpu.SemaphoreType.DMA for _ in range(2)],
        ],
        compiler_params=pltpu.CompilerParams(
            use_tc_tiling_on_sc=True, needs_layout_passes=False
        ),
    )
    def k(ids_hbm, table_hbm, out_hbm, ids_v, rows, obufs, rsem, osem):
        wid = lax.axis_index("s") * NC + lax.axis_index("c")
        f0 = wid * F_PER
        pltpu.sync_copy(ids_hbm, ids_v)

        def gather_half(rbuf, h, obuf):
            def per_b(bq, _):
                base = h * (HB * SEQ) + bq * SEQ
                for u in range(SEQ // 16):
                    idx = ids_v[pl.ds(base + u * 16, 16)]
                    obuf[bq, pl.ds(u * 16, 16)] = plsc.load_gather(rbuf, [idx])
                return _

            lax.fori_loop(0, HB, per_b, None)

        def process(fi, r):
            f = f0 + fi
            pltpu.make_async_copy(table_hbm.at[f0], rows[r], rsem[r]).wait()

            @pl.when(fi + 1 < F_PER)
            def _():
                pltpu.async_copy(
                    table_hbm.at[f + 1], rows[1 - r], rsem[1 - r]
                )

            for h in range(2):
                @pl.when(fi >= 1)
                def _():
                    pltpu.make_async_copy(
                        obufs[h], out_hbm.at[pl.ds(h * HB, HB), 0], osem[h]
                    ).wait()

                gather_half(rows[r], h, obufs[h])
                pltpu.async_copy(
                    obufs[h], out_hbm.at[pl.ds(h * HB, HB), f], osem[h]
                )

        pltpu.async_copy(table_hbm.at[f0], rows[0], rsem[0])

        def pair(j, _):
            process(j * 2, 0)
            process(j * 2 + 1, 1)
            return _

        lax.fori_loop(0, F_PER // 2, pair, None)
        for h in range(2):
            pltpu.make_async_copy(
                obufs[h], out_hbm.at[pl.ds(h * HB, HB), 0], osem[h]
            ).wait()

    return k(ids, table_t)


def kernel(input_ids, font_weights):
    ids = input_ids.reshape(-1).astype(jnp.int32)
    table_t = font_weights.T
    out_t = _sc_lookup(ids, table_t)
    return jnp.transpose(out_t, (0, 2, 1))


# speedup vs baseline: 7.3621x; 5.1144x over previous
import functools

import jax
import jax.numpy as jnp
from jax import lax
from jax.experimental import pallas as pl
from jax.experimental.pallas import tpu as pltpu
from jax.experimental.pallas import tpu_sc as plsc

V = 23236
D = 1728
BSZ = 64
SEQ = 512
NC = 2
NS = 16
NW = NC * NS
F_PER = D // NW
HB = BSZ // 2


@jax.jit
def _sc_lookup(ids, table_t):
    mesh = plsc.VectorSubcoreMesh(
        core_axis_name="c", subcore_axis_name="s", num_cores=NC, num_subcores=NS
    )

    @functools.partial(
        pl.kernel,
        out_type=jax.ShapeDtypeStruct((BSZ, D, SEQ), jnp.float32),
        mesh=mesh,
        scratch_types=[
            pltpu.VMEM((BSZ * SEQ,), jnp.int32),
            [pltpu.VMEM((V,), jnp.float32) for _ in range(2)],
            [pltpu.VMEM((HB, SEQ), jnp.float32) for _ in range(2)],
            [pltpu.SemaphoreType.DMA for _ in range(2)],
            [pltpu.SemaphoreType.DMA for _ in range(2)],
        ],
        compiler_params=pltpu.CompilerParams(
            use_tc_tiling_on_sc=True, needs_layout_passes=False
        ),
    )
    def k(ids_hbm, table_hbm, out_hbm, ids_v, rows, obufs, rsem, osem):
        wid = lax.axis_index("s") * NC + lax.axis_index("c")
        f0 = wid * F_PER
        pltpu.sync_copy(ids_hbm, ids_v)

        def gather_half(rbuf, h, obuf):
            def per_b(bq, _):
                base = h * (HB * SEQ) + bq * SEQ

                @plsc.parallel_loop(0, SEQ // 16, unroll=8)
                def _(u):
                    idx = ids_v[pl.ds(base + u * 16, 16)]
                    obuf[bq, pl.ds(u * 16, 16)] = plsc.load_gather(rbuf, [idx])

                return _

            lax.fori_loop(0, HB, per_b, None)

        def process(fi, r):
            f = f0 + fi
            pltpu.make_async_copy(table_hbm.at[f0], rows[r], rsem[r]).wait()

            @pl.when(fi + 1 < F_PER)
            def _():
                pltpu.async_copy(
                    table_hbm.at[f + 1], rows[1 - r], rsem[1 - r]
                )

            for h in range(2):
                @pl.when(fi >= 1)
                def _():
                    pltpu.make_async_copy(
                        obufs[h], out_hbm.at[pl.ds(h * HB, HB), 0], osem[h]
                    ).wait()

                gather_half(rows[r], h, obufs[h])
                pltpu.async_copy(
                    obufs[h], out_hbm.at[pl.ds(h * HB, HB), f], osem[h]
                )

        pltpu.async_copy(table_hbm.at[f0], rows[0], rsem[0])

        def pair(j, _):
            process(j * 2, 0)
            process(j * 2 + 1, 1)
            return _

        lax.fori_loop(0, F_PER // 2, pair, None)
        for h in range(2):
            pltpu.make_async_copy(
                obufs[h], out_hbm.at[pl.ds(h * HB, HB), 0], osem[h]
            ).wait()

    return k(ids, table_t)


def kernel(input_ids, font_weights):
    ids = input_ids.reshape(-1).astype(jnp.int32)
    table_t = font_weights.T
    out_t = _sc_lookup(ids, table_t)
    return jnp.transpose(out_t, (0, 2, 1))
